# initial kernel scaffold (unmeasured)
import jax
import jax.numpy as jnp
from jax import lax
from jax.experimental import pallas as pl
from jax.experimental.pallas import tpu as pltpu

N_DEV = 4


def kernel(x, w_mat):
    m_total, k_per = x.shape
    k_total, n = w_mat.shape
    m_per = m_total // N_DEV

    def body(x_ref, w_ref, out_ref, comm_ref, send_sems, recv_sems):
        my = lax.axis_index("i")

        barrier_sem = pltpu.get_barrier_semaphore()
        for d in range(1, N_DEV):
            pl.semaphore_signal(
                barrier_sem, inc=1,
                device_id=((my + d) % N_DEV,),
                device_id_type=pl.DeviceIdType.MESH,
            )
        pl.semaphore_wait(barrier_sem, N_DEV - 1)

        rdmas = []
        for d in range(1, N_DEV):
            peer = (my + d) % N_DEV
            rdma = pltpu.make_async_remote_copy(
                src_ref=x_ref.at[pl.ds(peer * m_per, m_per)],
                dst_ref=comm_ref.at[d - 1],
                send_sem=send_sems.at[d - 1],
                recv_sem=recv_sems.at[d - 1],
                device_id=(peer,),
                device_id_type=pl.DeviceIdType.MESH,
            )
            rdma.start()
            rdmas.append(rdma)

        out_ref[...] = jnp.dot(
            x_ref[pl.ds(my * m_per, m_per), :],
            w_ref[pl.ds(my * k_per, k_per), :],
            preferred_element_type=jnp.float32,
        )

        for d in (1, 3, 2):
            rdmas[d - 1].wait_recv()
            src = (my - d) % N_DEV
            out_ref[...] += jnp.dot(
                comm_ref[d - 1],
                w_ref[pl.ds(src * k_per, k_per), :],
                preferred_element_type=jnp.float32,
            )

        y = out_ref[...]
        c = 0.7978845608028654
        out_ref[...] = 0.5 * y * (1.0 + jnp.tanh(c * (y + 0.044715 * y * y * y)))

        for r in rdmas:
            r.wait_send()

    return pl.pallas_call(
        body,
        out_shape=jax.ShapeDtypeStruct((m_per, n), jnp.float32),
        in_specs=[
            pl.BlockSpec(memory_space=pltpu.VMEM),
            pl.BlockSpec(memory_space=pltpu.VMEM),
        ],
        out_specs=pl.BlockSpec(memory_space=pltpu.VMEM),
        scratch_shapes=[
            pltpu.VMEM((N_DEV - 1, m_per, k_per), x.dtype),
            pltpu.SemaphoreType.DMA((N_DEV - 1,)),
            pltpu.SemaphoreType.DMA((N_DEV - 1,)),
        ],
        compiler_params=pltpu.CompilerParams(collective_id=0),
    )(x, w_mat)


# baseline (device time: 72594 ns/iter reference)
import jax
import jax.numpy as jnp
from jax import lax
from jax.experimental import pallas as pl
from jax.experimental.pallas import tpu as pltpu

N_DEV = 4


def kernel(x, w_mat):
    m_total, k_per = x.shape
    k_total, n = w_mat.shape
    m_per = m_total // N_DEV

    def body(x_ref, w_ref, out_ref, send_buf, comm_ref, xloc_ref,
             wstage, w16_ref, send_sems, recv_sems, wdma_sems):
        my = lax.axis_index("i")

        order = [0, 1, 3, 2]

        def w_dma(t, slot):
            j = (my - order[t]) % N_DEV
            return pltpu.make_async_copy(
                w_ref.at[pl.ds(j * k_per, k_per)],
                wstage.at[slot],
                wdma_sems.at[slot],
            )

        w_dma(0, 0).start()
        w_dma(1, 1).start()

        barrier_sem = pltpu.get_barrier_semaphore()
        for d in range(1, N_DEV):
            pl.semaphore_signal(
                barrier_sem, inc=1,
                device_id=((my + d) % N_DEV,),
                device_id_type=pl.DeviceIdType.MESH,
            )
        pl.semaphore_wait(barrier_sem, N_DEV - 1)

        rdmas = {}
        for d in (1, 3, 2):
            peer = (my + d) % N_DEV
            send_buf[d - 1] = x_ref[pl.ds(peer * m_per, m_per), :].astype(
                jnp.bfloat16
            )
            rdma = pltpu.make_async_remote_copy(
                src_ref=send_buf.at[d - 1],
                dst_ref=comm_ref.at[d - 1],
                send_sem=send_sems.at[d - 1],
                recv_sem=recv_sems.at[d - 1],
                device_id=(peer,),
                device_id_type=pl.DeviceIdType.MESH,
            )
            rdma.start()
            rdmas[d] = rdma

        xloc_ref[...] = x_ref[pl.ds(my * m_per, m_per), :].astype(jnp.bfloat16)

        w_dma(0, 0).wait()
        w16_ref[...] = wstage[0].astype(jnp.bfloat16)
        w_dma(2, 0).start()
        out_ref[...] = jnp.dot(
            xloc_ref[...], w16_ref[...], preferred_element_type=jnp.float32
        )

        for t in (1, 2, 3):
            d = order[t]
            slot = t % 2
            rdmas[d].wait_recv()
            w_dma(t, slot).wait()
            w16_ref[...] = wstage[slot].astype(jnp.bfloat16)
            if t == 1:
                w_dma(3, 1).start()
            out_ref[...] += jnp.dot(
                comm_ref[d - 1], w16_ref[...],
                preferred_element_type=jnp.float32,
            )

        y = out_ref[...]
        c = 0.7978845608028654
        out_ref[...] = 0.5 * y * (1.0 + jnp.tanh(c * (y + 0.044715 * y * y * y)))

        for d in (1, 3, 2):
            rdmas[d].wait_send()

    return pl.pallas_call(
        body,
        out_shape=jax.ShapeDtypeStruct((m_per, n), jnp.float32),
        in_specs=[
            pl.BlockSpec(memory_space=pltpu.VMEM),
            pl.BlockSpec(memory_space=pl.ANY),
        ],
        out_specs=pl.BlockSpec(memory_space=pltpu.VMEM),
        scratch_shapes=[
            pltpu.VMEM((N_DEV - 1, m_per, k_per), jnp.bfloat16),
            pltpu.VMEM((N_DEV - 1, m_per, k_per), jnp.bfloat16),
            pltpu.VMEM((m_per, k_per), jnp.bfloat16),
            pltpu.VMEM((2, k_per, n), jnp.float32),
            pltpu.VMEM((k_per, n), jnp.bfloat16),
            pltpu.SemaphoreType.DMA((N_DEV - 1,)),
            pltpu.SemaphoreType.DMA((N_DEV - 1,)),
            pltpu.SemaphoreType.DMA((2,)),
        ],
        compiler_params=pltpu.CompilerParams(
            collective_id=0,
            vmem_limit_bytes=62 * 1024 * 1024,
        ),
    )(x, w_mat)


# device time: 69590 ns/iter; 1.0432x vs baseline; 1.0432x over previous
import jax
import jax.numpy as jnp
from jax import lax
from jax.experimental import pallas as pl
from jax.experimental.pallas import tpu as pltpu

N_DEV = 4
CHUNKS = 2


def kernel(x, w_mat):
    m_total, k_per = x.shape
    k_total, n = w_mat.shape
    m_per = m_total // N_DEV
    m_chunk = m_per // CHUNKS

    def body(x_ref, w_ref, out_ref, send_buf, comm_ref, xloc_ref,
             wstage, w16_ref, send_sems, recv_sems, wdma_sems):
        my = lax.axis_index("i")

        order = [0, 1, 3, 2]

        def w_dma(t, slot):
            j = (my - order[t]) % N_DEV
            return pltpu.make_async_copy(
                w_ref.at[pl.ds(j * k_per, k_per)],
                wstage.at[slot],
                wdma_sems.at[slot],
            )

        w_dma(0, 0).start()
        w_dma(1, 1).start()

        barrier_sem = pltpu.get_barrier_semaphore()
        for d in range(1, N_DEV):
            pl.semaphore_signal(
                barrier_sem, inc=1,
                device_id=((my + d) % N_DEV,),
                device_id_type=pl.DeviceIdType.MESH,
            )
        pl.semaphore_wait(barrier_sem, N_DEV - 1)

        def mk_rdma(d, c):
            peer = (my + d) % N_DEV
            return pltpu.make_async_remote_copy(
                src_ref=send_buf.at[d - 1, pl.ds(c * m_chunk, m_chunk)],
                dst_ref=comm_ref.at[d - 1, pl.ds(c * m_chunk, m_chunk)],
                send_sem=send_sems.at[(d - 1) * CHUNKS + c],
                recv_sem=recv_sems.at[(d - 1) * CHUNKS + c],
                device_id=(peer,),
                device_id_type=pl.DeviceIdType.MESH,
            )

        rdmas = {}
        for d in (1, 3, 2):
            send_buf[d - 1] = x_ref[
                pl.ds(((my + d) % N_DEV) * m_per, m_per), :
            ].astype(jnp.bfloat16)
            rdmas[(d, 0)] = mk_rdma(d, 0)
            rdmas[(d, 0)].start()
        for d in (1, 3, 2):
            for c in range(1, CHUNKS):
                rdmas[(d, c)] = mk_rdma(d, c)
                rdmas[(d, c)].start()

        xloc_ref[...] = x_ref[pl.ds(my * m_per, m_per), :].astype(jnp.bfloat16)

        w_dma(0, 0).wait()
        w16_ref[...] = wstage[0].astype(jnp.bfloat16)
        w_dma(2, 0).start()
        out_ref[...] = jnp.dot(
            xloc_ref[...], w16_ref[...], preferred_element_type=jnp.float32
        )

        c_gelu = 0.7978845608028654
        for t in (1, 2, 3):
            d = order[t]
            slot = t % 2
            w_dma(t, slot).wait()
            w16_ref[...] = wstage[slot].astype(jnp.bfloat16)
            if t == 1:
                w_dma(3, 1).start()
            for c in range(CHUNKS):
                rdmas[(d, c)].wait_recv()
                rows = pl.ds(c * m_chunk, m_chunk)
                acc = out_ref[rows, :] + jnp.dot(
                    comm_ref[d - 1, rows, :], w16_ref[...],
                    preferred_element_type=jnp.float32,
                )
                if t == 3:
                    acc = 0.5 * acc * (
                        1.0 + jnp.tanh(c_gelu * (acc + 0.044715 * acc * acc * acc))
                    )
                out_ref[rows, :] = acc

        for r in rdmas.values():
            r.wait_send()

    return pl.pallas_call(
        body,
        out_shape=jax.ShapeDtypeStruct((m_per, n), jnp.float32),
        in_specs=[
            pl.BlockSpec(memory_space=pltpu.VMEM),
            pl.BlockSpec(memory_space=pl.ANY),
        ],
        out_specs=pl.BlockSpec(memory_space=pltpu.VMEM),
        scratch_shapes=[
            pltpu.VMEM((N_DEV - 1, m_per, k_per), jnp.bfloat16),
            pltpu.VMEM((N_DEV - 1, m_per, k_per), jnp.bfloat16),
            pltpu.VMEM((m_per, k_per), jnp.bfloat16),
            pltpu.VMEM((2, k_per, n), jnp.float32),
            pltpu.VMEM((k_per, n), jnp.bfloat16),
            pltpu.SemaphoreType.DMA(((N_DEV - 1) * CHUNKS,)),
            pltpu.SemaphoreType.DMA(((N_DEV - 1) * CHUNKS,)),
            pltpu.SemaphoreType.DMA((2,)),
        ],
        compiler_params=pltpu.CompilerParams(
            collective_id=0,
            vmem_limit_bytes=62 * 1024 * 1024,
        ),
    )(x, w_mat)
